# Initial kernel scaffold; baseline (speedup 1.0000x reference)
import jax
import jax.numpy as jnp
from jax.experimental import pallas as pl


def kernel(context_vecs, hidden, trg_embs, vocab_dists, attn_dists, w_h, w_s, w_x, b_x, src_ids, pad_id):
    raise NotImplementedError("write your pallas kernel here")


# speedup vs baseline: 1.1052x; 1.1052x over previous
import jax
import jax.numpy as jnp
from jax import lax
from jax.experimental import pallas as pl
from jax.experimental.pallas import tpu as pltpu
from jax.experimental.pallas import tpu_sc as plsc

B, H, V, S = 1024, 128, 100000, 200
NCORES, NSUB, LANES = 2, 16, 16
NW = NCORES * NSUB
RPW = B // NW
GPR = 8
NGRP = RPW // GPR
CW = 2560
NCHUNK = 39
MAIN_COLS = CW * NCHUNK
TAILW = 256
NBUF = 4
TOT = NGRP * NCHUNK
KCH = (S + LANES - 1) // LANES


def _tc_gate(ctx_ref, hid_ref, trg_ref, attn_ref, wh_ref, ws_ref, wx_ref,
             bx_ref, pgen_ref, ad_ref):
    gen = (jnp.sum(ctx_ref[...] * wh_ref[...], axis=1, keepdims=True)
           + jnp.sum(hid_ref[...] * ws_ref[...], axis=1, keepdims=True)
           + jnp.sum(trg_ref[...] * wx_ref[...], axis=1, keepdims=True)
           + bx_ref[...])
    p = jax.nn.sigmoid(gen)
    pgen_ref[...] = p
    ad_ref[...] = (1.0 - p) * attn_ref[...]


def _tc_tail(outin_ref, vocab_ref, pgen_ref, tailp_ref, out_ref):
    del outin_ref
    out_ref[...] = pgen_ref[...] * vocab_ref[...] + tailp_ref[...]


def _perm(x, ind):
    return lax.gather(
        x, ind[:, None],
        lax.GatherDimensionNumbers(offset_dims=(), collapsed_slice_dims=(0,),
                                   start_index_map=(0,)),
        (1,), mode=lax.GatherScatterMode.PROMISE_IN_BOUNDS)


def _sc_body(vocab, pgen, ad, src, out, tailp, pgen_v, pgen_bc, idx_v, ad_v,
             cidx_v, cval_v, tail_v, buf, i0, i1, i2, i3, o0, o1, o2, o3):
    isems = [i0, i1, i2, i3]
    osems = [o0, o1, o2, o3]
    cid = lax.axis_index("c")
    sid = lax.axis_index("s")
    wid = sid * NCORES + cid
    base = wid * RPW
    iota = lax.iota(jnp.int32, LANES)

    def grpchunk(g):
        grp = g // NCHUNK
        ch = g - grp * NCHUNK
        return grp, ch

    def start_in(g, slot):
        grp, ch = grpchunk(g)
        pltpu.make_async_copy(
            vocab.at[pl.ds(base + grp * GPR, GPR), pl.ds(ch * CW, CW)],
            buf.at[slot], isems[slot]).start()

    def wait_in(slot):
        pltpu.make_async_copy(
            vocab.at[pl.ds(0, GPR), pl.ds(0, CW)], buf.at[slot],
            isems[slot]).wait()

    def start_out(g, slot):
        grp, ch = grpchunk(g)
        pltpu.make_async_copy(
            buf.at[slot],
            out.at[pl.ds(base + grp * GPR, GPR), pl.ds(ch * CW, CW)],
            osems[slot]).start()

    def wait_out(slot):
        pltpu.make_async_copy(
            buf.at[0], out.at[pl.ds(0, GPR), pl.ds(0, CW)],
            osems[slot]).wait()

    start_in(0, 0)
    start_in(1, 1)
    pltpu.sync_copy(pgen.at[pl.ds(base, RPW)], pgen_v)
    pltpu.sync_copy(src.at[pl.ds(base, RPW)], idx_v)
    pltpu.sync_copy(ad.at[pl.ds(base, RPW)], ad_v)

    @pl.loop(0, RPW)
    def _(r):
        pblk = pgen_v[pl.ds((r // LANES) * LANES, LANES)]
        pgen_bc[r, pl.ds(0, LANES)] = _perm(
            pblk, jnp.full((LANES,), r % LANES, dtype=jnp.int32))
        for t in range(TAILW // LANES):
            tail_v[r, pl.ds(t * LANES, LANES)] = jnp.zeros(
                (LANES,), jnp.float32)
        for k in range(KCH):
            off = k * LANES if k < KCH - 1 else S - LANES
            ii = idx_v[r, pl.ds(off, LANES)]
            vv = ad_v[r, pl.ds(off, LANES)]
            if k == KCH - 1:
                valid_from = KCH * LANES - S
                vv = jnp.where(iota >= valid_from, vv, 0.0)
            sk, sv = plsc.sort_key_val(ii, vv)
            pk = _perm(sk, jnp.maximum(iota - 1, 0))
            nk = _perm(sk, jnp.minimum(iota + 1, LANES - 1))
            is_first = (iota == 0) | (sk != pk)
            is_last = (iota == LANES - 1) | (sk != nk)
            csum = plsc.cumsum(sv)
            rstart = plsc.cummax(jnp.where(is_first, iota, 0))
            bval = jnp.where(rstart == 0, 0.0,
                             _perm(csum, jnp.maximum(rstart - 1, 0)))
            tot = csum - bval
            oidx = jnp.where(is_last, sk, V + iota)
            cidx_v[r, pl.ds(k * LANES, LANES)] = oidx
            cval_v[r, pl.ds(k * LANES, LANES)] = tot
            mt = is_last & (sk >= MAIN_COLS) & (sk < V)
            plsc.addupdate_scatter(
                tail_v, [jnp.full((LANES,), r, dtype=jnp.int32),
                         sk - MAIN_COLS], tot, mask=mt)

    pltpu.sync_copy(tail_v, tailp.at[pl.ds(base, RPW)])

    def compute(g, slot):
        grp, ch = grpchunk(g)
        lo = ch * CW
        for r8 in range(GPR):
            row = grp * GPR + r8
            pv = pgen_bc[row, pl.ds(0, LANES)]

            @plsc.parallel_loop(0, CW, step=LANES, unroll=8)
            def _(j):
                buf[slot, r8, pl.ds(j, LANES)] = (
                    buf[slot, r8, pl.ds(j, LANES)] * pv)

        for r8 in range(GPR):
            row = grp * GPR + r8
            for k in range(KCH):
                ii = cidx_v[row, pl.ds(k * LANES, LANES)]
                vv = cval_v[row, pl.ds(k * LANES, LANES)]
                m = (ii >= lo) & (ii < lo + CW)
                plsc.addupdate_scatter(
                    buf, [jnp.full((LANES,), slot, dtype=jnp.int32),
                          jnp.full((LANES,), r8, dtype=jnp.int32),
                          ii - lo], vv, mask=m)

    @pl.loop(0, TOT, step=NBUF)
    def _(g0):
        for b in range(NBUF):
            g = g0 + b

            @pl.when(g >= 2)
            def _():
                wait_out((b + 2) % NBUF)

            @pl.when(g + 2 < TOT)
            def _():
                start_in(g + 2, (b + 2) % NBUF)

            @pl.when(g < TOT)
            def _():
                wait_in(b)
                compute(g, b)
                start_out(g, b)

    wait_out((TOT - 2) % NBUF)
    wait_out((TOT - 1) % NBUF)


def kernel(context_vecs, hidden, trg_embs, vocab_dists, attn_dists, w_h, w_s,
           w_x, b_x, src_ids, pad_id):
    del pad_id
    gate = pl.pallas_call(
        _tc_gate,
        out_shape=(jax.ShapeDtypeStruct((B, 1), jnp.float32),
                   jax.ShapeDtypeStruct((B, S), jnp.float32)),
    )
    pgen2, ad = gate(context_vecs, hidden, trg_embs, attn_dists, w_h, w_s,
                     w_x, b_x.reshape(1, 1))
    pgen = pgen2.reshape(B)

    mesh = plsc.VectorSubcoreMesh(core_axis_name="c", subcore_axis_name="s")
    sc_fn = pl.kernel(
        _sc_body,
        out_type=(jax.ShapeDtypeStruct((B, V), jnp.float32),
                  jax.ShapeDtypeStruct((B, TAILW), jnp.float32)),
        mesh=mesh,
        compiler_params=pltpu.CompilerParams(needs_layout_passes=False),
        scratch_types=[
            pltpu.VMEM((RPW,), jnp.float32),
            pltpu.VMEM((RPW, LANES), jnp.float32),
            pltpu.VMEM((RPW, S), jnp.int32),
            pltpu.VMEM((RPW, S), jnp.float32),
            pltpu.VMEM((RPW, KCH * LANES), jnp.int32),
            pltpu.VMEM((RPW, KCH * LANES), jnp.float32),
            pltpu.VMEM((RPW, TAILW), jnp.float32),
            pltpu.VMEM((NBUF, GPR, CW), jnp.float32),
            pltpu.SemaphoreType.DMA,
            pltpu.SemaphoreType.DMA,
            pltpu.SemaphoreType.DMA,
            pltpu.SemaphoreType.DMA,
            pltpu.SemaphoreType.DMA,
            pltpu.SemaphoreType.DMA,
            pltpu.SemaphoreType.DMA,
            pltpu.SemaphoreType.DMA,
        ],
    )
    out1, tailp = sc_fn(vocab_dists, pgen, ad, src_ids.astype(jnp.int32))

    nblk = 8
    rb = B // nblk
    tail_fix = pl.pallas_call(
        _tc_tail,
        grid=(nblk,),
        in_specs=[
            pl.BlockSpec((rb, TAILW), lambda i: (i, MAIN_COLS // TAILW)),
            pl.BlockSpec((rb, TAILW), lambda i: (i, MAIN_COLS // TAILW)),
            pl.BlockSpec((rb, 1), lambda i: (i, 0)),
            pl.BlockSpec((rb, TAILW), lambda i: (i, 0)),
        ],
        out_specs=pl.BlockSpec((rb, TAILW), lambda i: (i, MAIN_COLS // TAILW)),
        out_shape=jax.ShapeDtypeStruct((B, V), jnp.float32),
        input_output_aliases={0: 0},
    )
    return tail_fix(out1, vocab_dists, pgen2, tailp)
